# baseline (speedup 1.0000x reference)
import functools

import jax
import jax.numpy as jnp
from jax import lax
from jax.experimental import pallas as pl
from jax.experimental.pallas import tpu as pltpu
from jax.experimental.pallas import tpu_sc as plsc

N = 10000
D = 128
E = 320000
NC, NS, L = 2, 16, 16
NW = NC * NS
C = 80
EPT = 10240
NCH = EPT // C
RPT = 640
SR = NS * RPT
ZR = 16


def _sc_segment_sum(x, src_r, dst_r):
    mesh = plsc.VectorSubcoreMesh(core_axis_name="c", subcore_axis_name="s")

    @functools.partial(
        pl.kernel,
        out_type=jax.ShapeDtypeStruct((NC, SR, D), jnp.float32),
        mesh=mesh,
        scratch_types=[
            pltpu.VMEM((NCH, C), jnp.int32),
            pltpu.VMEM((NCH, C), jnp.int32),
            pltpu.VMEM((2, C, D), jnp.float32),
            pltpu.VMEM((ZR, D), jnp.float32),
            pltpu.VMEM_SHARED((SR, D), jnp.float32),
            pltpu.SemaphoreType.DMA((2,)),
            pltpu.SemaphoreType.DMA((2,)),
            pltpu.SemaphoreType.DMA,
        ],
        compiler_params=pltpu.CompilerParams(use_tc_tiling_on_sc=False),
    )
    def seg_kernel(x_hbm, src_hbm, dst_hbm, out_hbm,
                   src_v, dst_v, gbuf, zbuf, acc_sh, gsem, ssem, zsem):
        cid = lax.axis_index("c")
        sid = lax.axis_index("s")
        wid = cid * NS + sid

        pltpu.sync_copy(src_hbm.at[wid], src_v)
        pltpu.sync_copy(dst_hbm.at[wid], dst_v)

        zv = jnp.zeros((L,), jnp.float32)

        @pl.loop(0, ZR)
        def _(r):
            @pl.loop(0, D, step=L)
            def _(cc):
                zbuf[r, pl.ds(cc, L)] = zv

        base = sid * RPT

        @pl.loop(0, RPT, step=ZR)
        def _(o):
            pltpu.async_copy(zbuf, acc_sh.at[pl.ds(base + o, ZR)], zsem)

        @pl.loop(0, RPT, step=ZR)
        def _(o):
            pltpu.make_async_copy(zbuf, acc_sh.at[pl.ds(base, ZR)], zsem).wait()

        plsc.subcore_barrier()

        pltpu.async_copy(x_hbm.at[src_v.at[0]], gbuf.at[0], gsem.at[0])

        @pl.loop(0, NCH, step=2)
        def _(g0):
            for h in range(2):
                c = g0 + h
                oth = 1 - h
                pltpu.make_async_copy(x_hbm.at[src_v.at[c]], gbuf.at[h],
                                      gsem.at[h]).wait()

                @pl.when(c + 1 < NCH)
                def _():
                    pltpu.async_copy(x_hbm.at[src_v.at[c + 1]],
                                     gbuf.at[oth], gsem.at[oth])

                pltpu.async_copy(gbuf.at[h], acc_sh.at[dst_v.at[c]],
                                 ssem.at[h], add=True)
                pltpu.make_async_copy(gbuf.at[h], acc_sh.at[dst_v.at[c]],
                                      ssem.at[h]).wait()

        plsc.subcore_barrier()

        pltpu.sync_copy(acc_sh.at[pl.ds(base, RPT)],
                        out_hbm.at[cid, pl.ds(base, RPT)])

    return seg_kernel(x, src_r, dst_r)


def _tc_head(x, parts, W_self, W_nbr, b2, w2):
    R = 1000
    G = N // R

    def head_kernel(x_ref, p_ref, ws_ref, wn_ref, b_ref, w_ref, o_ref):
        agg = p_ref[0] + p_ref[1]
        h = jnp.dot(x_ref[...], ws_ref[...],
                    preferred_element_type=jnp.float32,
                    precision=lax.Precision.HIGHEST)
        h = h + jnp.dot(agg, wn_ref[...],
                        preferred_element_type=jnp.float32,
                        precision=lax.Precision.HIGHEST)
        h = jnp.maximum(h + b_ref[...], 0.0)
        o_ref[...] = jnp.sum(h * w_ref[...], axis=1, keepdims=True)

    return pl.pallas_call(
        head_kernel,
        grid=(G,),
        in_specs=[
            pl.BlockSpec((R, D), lambda i: (i, 0)),
            pl.BlockSpec((NC, R, D), lambda i: (0, i, 0)),
            pl.BlockSpec((D, D), lambda i: (0, 0)),
            pl.BlockSpec((D, D), lambda i: (0, 0)),
            pl.BlockSpec((1, D), lambda i: (0, 0)),
            pl.BlockSpec((1, D), lambda i: (0, 0)),
        ],
        out_specs=pl.BlockSpec((R, 1), lambda i: (i, 0)),
        out_shape=jax.ShapeDtypeStruct((N, 1), jnp.float32),
    )(x, parts, W_self, W_nbr, b2, w2)


@jax.jit
def kernel(x, edge_index, W_self, W_nbr, b, w_out):
    src = edge_index[0]
    dst = edge_index[1]
    pad = NW * EPT - E
    src_r = jnp.concatenate([src, jnp.zeros((pad,), jnp.int32)]).reshape(NW, NCH, C)
    dst_r = jnp.concatenate([dst, jnp.full((pad,), N, jnp.int32)]).reshape(NW, NCH, C)
    parts = _sc_segment_sum(x, src_r, dst_r)
    out = _tc_head(x, parts, W_self, W_nbr,
                   b.reshape(1, D), w_out.reshape(1, D))
    return out[:, 0]


# speedup vs baseline: 1.4461x; 1.4461x over previous
import functools

import jax
import jax.numpy as jnp
from jax import lax
from jax.experimental import pallas as pl
from jax.experimental.pallas import tpu as pltpu
from jax.experimental.pallas import tpu_sc as plsc

N = 10000
D = 128
DW = D // 2
E = 320000
NC, NS, L = 2, 16, 16
C = 128
EPT = 20480
NCH = EPT // C
RPT = 640
SR = NS * RPT
ZR = 16
NBUF = 5
NGRP = NCH // NBUF


def _sc_segment_sum(xs, src_r2, dst_r):
    mesh = plsc.VectorSubcoreMesh(core_axis_name="c", subcore_axis_name="s")

    @functools.partial(
        pl.kernel,
        out_type=jax.ShapeDtypeStruct((NC, SR, DW), jnp.float32),
        mesh=mesh,
        scratch_types=[
            pltpu.VMEM((NCH, C), jnp.int32),
            pltpu.VMEM((NCH, C), jnp.int32),
            pltpu.VMEM((NBUF, C, DW), jnp.float32),
            pltpu.VMEM((ZR, DW), jnp.float32),
            pltpu.VMEM_SHARED((SR, DW), jnp.float32),
            pltpu.SemaphoreType.DMA((NBUF,)),
            pltpu.SemaphoreType.DMA((NBUF,)),
            pltpu.SemaphoreType.DMA,
        ],
        compiler_params=pltpu.CompilerParams(use_tc_tiling_on_sc=False),
    )
    def seg_kernel(xs_hbm, src_hbm, dst_hbm, out_hbm,
                   src_v, dst_v, gbuf, zbuf, acc_sh, gsem, ssem, zsem):
        cid = lax.axis_index("c")
        sid = lax.axis_index("s")

        pltpu.sync_copy(src_hbm.at[cid, sid], src_v)
        pltpu.sync_copy(dst_hbm.at[sid], dst_v)

        zv = jnp.zeros((L,), jnp.float32)

        @pl.loop(0, ZR)
        def _(r):
            @pl.loop(0, DW, step=L)
            def _(cc):
                zbuf[r, pl.ds(cc, L)] = zv

        base = sid * RPT

        @pl.loop(0, RPT, step=ZR)
        def _(o):
            pltpu.async_copy(zbuf, acc_sh.at[pl.ds(base + o, ZR)], zsem)

        @pl.loop(0, RPT, step=ZR)
        def _(o):
            pltpu.make_async_copy(zbuf, acc_sh.at[pl.ds(base, ZR)], zsem).wait()

        plsc.subcore_barrier()

        for bb in range(NBUF):
            pltpu.async_copy(xs_hbm.at[src_v.at[bb]], gbuf.at[bb], gsem.at[bb])

        @pl.loop(0, NGRP)
        def _(g):
            c0 = g * NBUF
            for bb in range(NBUF):
                c = c0 + bb
                pltpu.make_async_copy(xs_hbm.at[src_v.at[c]], gbuf.at[bb],
                                      gsem.at[bb]).wait()
                pltpu.async_copy(gbuf.at[bb], acc_sh.at[dst_v.at[c]],
                                 ssem.at[bb], add=True)
            for bb in range(NBUF):
                c = c0 + bb
                pltpu.make_async_copy(gbuf.at[bb], acc_sh.at[dst_v.at[c]],
                                      ssem.at[bb]).wait()

                @pl.when(c + NBUF < NCH)
                def _():
                    pltpu.async_copy(xs_hbm.at[src_v.at[c + NBUF]],
                                     gbuf.at[bb], gsem.at[bb])

        plsc.subcore_barrier()

        pltpu.sync_copy(acc_sh.at[pl.ds(base, RPT)],
                        out_hbm.at[cid, pl.ds(base, RPT)])

    return seg_kernel(xs, src_r2, dst_r)


def _tc_head(x, parts, W_self, W_nbr, b2, w2):
    R = 1000
    G = N // R

    def head_kernel(x_ref, p_ref, ws_ref, wn_ref, b_ref, w_ref, o_ref):
        agg = jnp.concatenate([p_ref[0], p_ref[1]], axis=-1)
        h = jnp.dot(x_ref[...], ws_ref[...],
                    preferred_element_type=jnp.float32,
                    precision=lax.Precision.HIGHEST)
        h = h + jnp.dot(agg, wn_ref[...],
                        preferred_element_type=jnp.float32,
                        precision=lax.Precision.HIGHEST)
        h = jnp.maximum(h + b_ref[...], 0.0)
        o_ref[...] = jnp.sum(h * w_ref[...], axis=1, keepdims=True)

    return pl.pallas_call(
        head_kernel,
        grid=(G,),
        in_specs=[
            pl.BlockSpec((R, D), lambda i: (i, 0)),
            pl.BlockSpec((NC, R, DW), lambda i: (0, i, 0)),
            pl.BlockSpec((D, D), lambda i: (0, 0)),
            pl.BlockSpec((D, D), lambda i: (0, 0)),
            pl.BlockSpec((1, D), lambda i: (0, 0)),
            pl.BlockSpec((1, D), lambda i: (0, 0)),
        ],
        out_specs=pl.BlockSpec((R, 1), lambda i: (i, 0)),
        out_shape=jax.ShapeDtypeStruct((N, 1), jnp.float32),
    )(x, parts, W_self, W_nbr, b2, w2)


@jax.jit
def kernel(x, edge_index, W_self, W_nbr, b, w_out):
    src = edge_index[0]
    dst = edge_index[1]
    pad = NS * EPT - E
    src_r = jnp.concatenate([src, jnp.zeros((pad,), jnp.int32)]).reshape(NS, NCH, C)
    src_r2 = src_r[None] + (jnp.arange(NC, dtype=jnp.int32) * N)[:, None, None, None]
    dst_r = jnp.concatenate([dst, jnp.full((pad,), N, jnp.int32)]).reshape(NS, NCH, C)
    xs = jnp.concatenate([x[:, :DW], x[:, DW:]], axis=0)
    parts = _sc_segment_sum(xs, src_r2, dst_r)
    out = _tc_head(x, parts, W_self, W_nbr,
                   b.reshape(1, D), w_out.reshape(1, D))
    return out[:, 0]


# speedup vs baseline: 3.0577x; 2.1145x over previous
import functools

import jax
import jax.numpy as jnp
from jax import lax
from jax.experimental import pallas as pl
from jax.experimental.pallas import tpu as pltpu
from jax.experimental.pallas import tpu_sc as plsc

N = 10000
D = 128
DW = D // 2
E = 320000
NC, NS, L = 2, 16, 16
C = 125
EPT = E // NS
NCH = EPT // C
RPT = N // NS
ZR = 25
NBUF = 5
NGRP = NCH // NBUF


def _sc_segment_sum(xa, xb, er):
    mesh = plsc.VectorSubcoreMesh(core_axis_name="c", subcore_axis_name="s")

    @functools.partial(
        pl.kernel,
        out_type=jax.ShapeDtypeStruct((NC, N, DW), jnp.float32),
        mesh=mesh,
        scratch_types=[
            pltpu.VMEM((NCH, C), jnp.int32),
            pltpu.VMEM((NCH, C), jnp.int32),
            pltpu.VMEM((NBUF, C, DW), jnp.float32),
            pltpu.VMEM((ZR, DW), jnp.float32),
            pltpu.VMEM_SHARED((N, DW), jnp.float32),
            pltpu.SemaphoreType.DMA((NBUF,)),
            pltpu.SemaphoreType.DMA((NBUF,)),
            pltpu.SemaphoreType.DMA,
        ],
        compiler_params=pltpu.CompilerParams(use_tc_tiling_on_sc=False),
    )
    def seg_kernel(xa_hbm, xb_hbm, er_hbm, out_hbm,
                   src_v, dst_v, gbuf, zbuf, acc_sh, gsem, ssem, zsem):
        cid = lax.axis_index("c")
        sid = lax.axis_index("s")

        pltpu.sync_copy(er_hbm.at[0, sid], src_v)
        pltpu.sync_copy(er_hbm.at[1, sid], dst_v)

        zv = jnp.zeros((L,), jnp.float32)

        @pl.loop(0, ZR)
        def _(r):
            @pl.loop(0, DW, step=L)
            def _(cc):
                zbuf[r, pl.ds(cc, L)] = zv

        base = sid * RPT

        @pl.loop(0, RPT, step=ZR)
        def _(o):
            pltpu.async_copy(zbuf, acc_sh.at[pl.ds(base + o, ZR)], zsem)

        @pl.loop(0, RPT, step=ZR)
        def _(o):
            pltpu.make_async_copy(zbuf, acc_sh.at[pl.ds(base, ZR)], zsem).wait()

        plsc.subcore_barrier()

        def ring(xcol):
            for bb in range(NBUF):
                pltpu.async_copy(xcol.at[src_v.at[bb]], gbuf.at[bb],
                                 gsem.at[bb])

            @pl.loop(0, NGRP)
            def _(g):
                c0 = g * NBUF
                for bb in range(NBUF):
                    c = c0 + bb
                    pltpu.make_async_copy(xcol.at[src_v.at[c]], gbuf.at[bb],
                                          gsem.at[bb]).wait()
                    pltpu.async_copy(gbuf.at[bb], acc_sh.at[dst_v.at[c]],
                                     ssem.at[bb], add=True)
                for bb in range(NBUF):
                    c = c0 + bb
                    pltpu.make_async_copy(gbuf.at[bb], acc_sh.at[dst_v.at[c]],
                                          ssem.at[bb]).wait()

                    @pl.when(c + NBUF < NCH)
                    def _():
                        pltpu.async_copy(xcol.at[src_v.at[c + NBUF]],
                                         gbuf.at[bb], gsem.at[bb])

        @pl.when(cid == 0)
        def _():
            ring(xa_hbm)

        @pl.when(cid == 1)
        def _():
            ring(xb_hbm)

        plsc.subcore_barrier()

        pltpu.sync_copy(acc_sh.at[pl.ds(base, RPT)],
                        out_hbm.at[cid, pl.ds(base, RPT)])

    return seg_kernel(xa, xb, er)


def _tc_head(x, parts, W_self, W_nbr, b2, w2):
    R = 1000
    G = N // R

    def head_kernel(x_ref, p_ref, ws_ref, wn_ref, b_ref, w_ref, o_ref):
        agg = jnp.concatenate([p_ref[0], p_ref[1]], axis=-1)
        h = jnp.dot(x_ref[...], ws_ref[...],
                    preferred_element_type=jnp.float32,
                    precision=lax.Precision.HIGHEST)
        h = h + jnp.dot(agg, wn_ref[...],
                        preferred_element_type=jnp.float32,
                        precision=lax.Precision.HIGHEST)
        h = jnp.maximum(h + b_ref[...], 0.0)
        o_ref[...] = jnp.sum(h * w_ref[...], axis=1, keepdims=True)

    return pl.pallas_call(
        head_kernel,
        grid=(G,),
        in_specs=[
            pl.BlockSpec((R, D), lambda i: (i, 0)),
            pl.BlockSpec((NC, R, DW), lambda i: (0, i, 0)),
            pl.BlockSpec((D, D), lambda i: (0, 0)),
            pl.BlockSpec((D, D), lambda i: (0, 0)),
            pl.BlockSpec((1, D), lambda i: (0, 0)),
            pl.BlockSpec((1, D), lambda i: (0, 0)),
        ],
        out_specs=pl.BlockSpec((R, 1), lambda i: (i, 0)),
        out_shape=jax.ShapeDtypeStruct((N, 1), jnp.float32),
    )(x, parts, W_self, W_nbr, b2, w2)


@jax.jit
def kernel(x, edge_index, W_self, W_nbr, b, w_out):
    er = edge_index.reshape(2, NS, NCH, C)
    parts = _sc_segment_sum(x[:, :DW], x[:, DW:], er)
    out = _tc_head(x, parts, W_self, W_nbr,
                   b.reshape(1, D), w_out.reshape(1, D))
    return out[:, 0]


# speedup vs baseline: 3.2081x; 1.0492x over previous
import functools

import jax
import jax.numpy as jnp
from jax import lax
from jax.experimental import pallas as pl
from jax.experimental.pallas import tpu as pltpu
from jax.experimental.pallas import tpu_sc as plsc

N = 10000
D = 128
DW = D // 2
E = 320000
NC, NS, L = 2, 16, 16
C = 125
EPT = E // NS
NCH = EPT // C
RPT = N // NS
ZR = 25
NBUF = 5
NGRP = NCH // NBUF


def _sc_segment_sum(xa, xb, er):
    mesh = plsc.VectorSubcoreMesh(core_axis_name="c", subcore_axis_name="s")

    @functools.partial(
        pl.kernel,
        out_type=jax.ShapeDtypeStruct((NC, N, DW), jnp.float32),
        mesh=mesh,
        scratch_types=[
            pltpu.VMEM((NCH, C), jnp.int32),
            pltpu.VMEM((NCH, C), jnp.int32),
            pltpu.VMEM((NBUF, C, DW), jnp.float32),
            pltpu.VMEM((ZR, DW), jnp.float32),
            pltpu.VMEM_SHARED((N, DW), jnp.float32),
            pltpu.SemaphoreType.DMA((NBUF,)),
            pltpu.SemaphoreType.DMA((NBUF,)),
            pltpu.SemaphoreType.DMA,
        ],
        compiler_params=pltpu.CompilerParams(use_tc_tiling_on_sc=False),
    )
    def seg_kernel(xa_hbm, xb_hbm, er_hbm, out_hbm,
                   src_v, dst_v, gbuf, zbuf, acc_sh, gsem, ssem, zsem):
        cid = lax.axis_index("c")
        sid = lax.axis_index("s")

        pltpu.sync_copy(er_hbm.at[0, sid], src_v)
        pltpu.sync_copy(er_hbm.at[1, sid], dst_v)

        zv = jnp.zeros((L,), jnp.float32)

        @pl.loop(0, ZR)
        def _(r):
            @pl.loop(0, DW, step=L)
            def _(cc):
                zbuf[r, pl.ds(cc, L)] = zv

        base = sid * RPT

        @pl.loop(0, RPT, step=ZR)
        def _(o):
            pltpu.async_copy(zbuf, acc_sh.at[pl.ds(base + o, ZR)], zsem)

        @pl.loop(0, RPT, step=ZR)
        def _(o):
            pltpu.make_async_copy(zbuf, acc_sh.at[pl.ds(base, ZR)], zsem).wait()

        plsc.subcore_barrier()

        def ring(xcol):
            for bb in range(NBUF):
                pltpu.async_copy(xcol.at[src_v.at[bb]], gbuf.at[bb],
                                 gsem.at[bb])

            @pl.loop(0, NGRP)
            def _(g):
                c0 = g * NBUF
                for bb in range(NBUF):
                    c = c0 + bb
                    pltpu.make_async_copy(xcol.at[src_v.at[c]], gbuf.at[bb],
                                          gsem.at[bb]).wait()
                    pltpu.async_copy(gbuf.at[bb], acc_sh.at[dst_v.at[c]],
                                     ssem.at[bb], add=True)
                for bb in range(NBUF):
                    c = c0 + bb
                    pltpu.make_async_copy(gbuf.at[bb], acc_sh.at[dst_v.at[c]],
                                          ssem.at[bb]).wait()

                    @pl.when(c + NBUF < NCH)
                    def _():
                        pltpu.async_copy(xcol.at[src_v.at[c + NBUF]],
                                         gbuf.at[bb], gsem.at[bb])

        @pl.when(cid == 0)
        def _():
            ring(xa_hbm)

        @pl.when(cid == 1)
        def _():
            ring(xb_hbm)

        plsc.subcore_barrier()

        pltpu.sync_copy(acc_sh.at[pl.ds(base, RPT)],
                        out_hbm.at[cid, pl.ds(base, RPT)])

    return seg_kernel(xa, xb, er)


def _tc_self(x, W_self, b2):
    R = 1000
    G = N // R

    def self_kernel(x_ref, ws_ref, b_ref, o_ref):
        o_ref[...] = jnp.dot(x_ref[...], ws_ref[...],
                             preferred_element_type=jnp.float32) + b_ref[...]

    return pl.pallas_call(
        self_kernel,
        grid=(G,),
        in_specs=[
            pl.BlockSpec((R, D), lambda i: (i, 0)),
            pl.BlockSpec((D, D), lambda i: (0, 0)),
            pl.BlockSpec((1, D), lambda i: (0, 0)),
        ],
        out_specs=pl.BlockSpec((R, D), lambda i: (i, 0)),
        out_shape=jax.ShapeDtypeStruct((N, D), jnp.float32),
    )(x, W_self, b2)


def _tc_head(ha, parts, W_nbr, w2):
    R = 1000
    G = N // R

    def head_kernel(ha_ref, p_ref, wn_ref, w_ref, o_ref):
        agg = jnp.concatenate([p_ref[0], p_ref[1]], axis=-1)
        h = ha_ref[...] + jnp.dot(agg, wn_ref[...],
                                  preferred_element_type=jnp.float32)
        h = jnp.maximum(h, 0.0)
        o_ref[...] = jnp.sum(h * w_ref[...], axis=1, keepdims=True)

    return pl.pallas_call(
        head_kernel,
        grid=(G,),
        in_specs=[
            pl.BlockSpec((R, D), lambda i: (i, 0)),
            pl.BlockSpec((NC, R, DW), lambda i: (0, i, 0)),
            pl.BlockSpec((D, D), lambda i: (0, 0)),
            pl.BlockSpec((1, D), lambda i: (0, 0)),
        ],
        out_specs=pl.BlockSpec((R, 1), lambda i: (i, 0)),
        out_shape=jax.ShapeDtypeStruct((N, 1), jnp.float32),
    )(ha, parts, W_nbr, w2)


@jax.jit
def kernel(x, edge_index, W_self, W_nbr, b, w_out):
    er = edge_index.reshape(2, NS, NCH, C)
    parts = _sc_segment_sum(x[:, :DW], x[:, DW:], er)
    ha = _tc_self(x, W_self, b.reshape(1, D))
    out = _tc_head(ha, parts, W_nbr, w_out.reshape(1, D))
    return out[:, 0]


# speedup vs baseline: 3.2532x; 1.0141x over previous
import functools

import jax
import jax.numpy as jnp
from jax import lax
from jax.experimental import pallas as pl
from jax.experimental.pallas import tpu as pltpu
from jax.experimental.pallas import tpu_sc as plsc

N = 10000
D = 128
DW = D // 2
E = 320000
NC, NS, L = 2, 16, 16
C = 125
EPT = E // NS
NCH = EPT // C
RPT = N // NS
ZR = 125
NBUF = 5
NGRP = NCH // NBUF


def _sc_segment_sum(xa, xb, er):
    mesh = plsc.VectorSubcoreMesh(core_axis_name="c", subcore_axis_name="s")

    @functools.partial(
        pl.kernel,
        out_type=jax.ShapeDtypeStruct((NC, N, DW), jnp.float32),
        mesh=mesh,
        scratch_types=[
            pltpu.VMEM((NCH, C), jnp.int32),
            pltpu.VMEM((NCH, C), jnp.int32),
            pltpu.VMEM((NBUF, C, DW), jnp.float32),
            pltpu.VMEM((ZR, DW), jnp.float32),
            pltpu.VMEM_SHARED((N, DW), jnp.float32),
            pltpu.SemaphoreType.DMA((NBUF,)),
            pltpu.SemaphoreType.DMA((NBUF,)),
            pltpu.SemaphoreType.DMA,
        ],
        compiler_params=pltpu.CompilerParams(use_tc_tiling_on_sc=False),
    )
    def seg_kernel(xa_hbm, xb_hbm, er_hbm, out_hbm,
                   src_v, dst_v, gbuf, zbuf, acc_sh, gsem, ssem, zsem):
        cid = lax.axis_index("c")
        sid = lax.axis_index("s")

        cp_src = pltpu.async_copy(er_hbm.at[0, sid], src_v, gsem.at[0])
        cp_dst = pltpu.async_copy(er_hbm.at[1, sid], dst_v, gsem.at[1])

        zv = jnp.zeros((L,), jnp.float32)

        @pl.loop(0, ZR)
        def _(r):
            @pl.loop(0, DW, step=L)
            def _(cc):
                zbuf[r, pl.ds(cc, L)] = zv

        base = sid * RPT

        @pl.loop(0, RPT, step=ZR)
        def _(o):
            pltpu.async_copy(zbuf, acc_sh.at[pl.ds(base + o, ZR)], zsem)

        @pl.loop(0, RPT, step=ZR)
        def _(o):
            pltpu.make_async_copy(zbuf, acc_sh.at[pl.ds(base, ZR)], zsem).wait()

        cp_src.wait()
        cp_dst.wait()
        plsc.subcore_barrier()

        def ring(xcol):
            for bb in range(NBUF):
                pltpu.async_copy(xcol.at[src_v.at[bb]], gbuf.at[bb],
                                 gsem.at[bb])

            @pl.loop(0, NGRP)
            def _(g):
                c0 = g * NBUF
                for bb in range(NBUF):
                    c = c0 + bb
                    pltpu.make_async_copy(xcol.at[src_v.at[c]], gbuf.at[bb],
                                          gsem.at[bb]).wait()
                    pltpu.async_copy(gbuf.at[bb], acc_sh.at[dst_v.at[c]],
                                     ssem.at[bb], add=True)
                for bb in range(NBUF):
                    c = c0 + bb
                    pltpu.make_async_copy(gbuf.at[bb], acc_sh.at[dst_v.at[c]],
                                          ssem.at[bb]).wait()

                    @pl.when(c + NBUF < NCH)
                    def _():
                        pltpu.async_copy(xcol.at[src_v.at[c + NBUF]],
                                         gbuf.at[bb], gsem.at[bb])

        @pl.when(cid == 0)
        def _():
            ring(xa_hbm)

        @pl.when(cid == 1)
        def _():
            ring(xb_hbm)

        plsc.subcore_barrier()

        pltpu.sync_copy(acc_sh.at[pl.ds(base, RPT)],
                        out_hbm.at[cid, pl.ds(base, RPT)])

    return seg_kernel(xa, xb, er)


def _tc_self(x, W_self, b2):
    R = 1000
    G = N // R

    def self_kernel(x_ref, ws_ref, b_ref, o_ref):
        o_ref[...] = jnp.dot(x_ref[...], ws_ref[...],
                             preferred_element_type=jnp.float32) + b_ref[...]

    return pl.pallas_call(
        self_kernel,
        grid=(G,),
        in_specs=[
            pl.BlockSpec((R, D), lambda i: (i, 0)),
            pl.BlockSpec((D, D), lambda i: (0, 0)),
            pl.BlockSpec((1, D), lambda i: (0, 0)),
        ],
        out_specs=pl.BlockSpec((R, D), lambda i: (i, 0)),
        out_shape=jax.ShapeDtypeStruct((N, D), jnp.float32),
    )(x, W_self, b2)


def _tc_head(ha, parts, W_nbr, w2):
    R = 1000
    G = N // R

    def head_kernel(ha_ref, p_ref, wn_ref, w_ref, o_ref):
        agg = jnp.concatenate([p_ref[0], p_ref[1]], axis=-1)
        h = ha_ref[...] + jnp.dot(agg, wn_ref[...],
                                  preferred_element_type=jnp.float32)
        h = jnp.maximum(h, 0.0)
        o_ref[...] = jnp.sum(h * w_ref[...], axis=1, keepdims=True)

    return pl.pallas_call(
        head_kernel,
        grid=(G,),
        in_specs=[
            pl.BlockSpec((R, D), lambda i: (i, 0)),
            pl.BlockSpec((NC, R, DW), lambda i: (0, i, 0)),
            pl.BlockSpec((D, D), lambda i: (0, 0)),
            pl.BlockSpec((1, D), lambda i: (0, 0)),
        ],
        out_specs=pl.BlockSpec((R, 1), lambda i: (i, 0)),
        out_shape=jax.ShapeDtypeStruct((N, 1), jnp.float32),
    )(ha, parts, W_nbr, w2)


@jax.jit
def kernel(x, edge_index, W_self, W_nbr, b, w_out):
    er = edge_index.reshape(2, NS, NCH, C)
    parts = _sc_segment_sum(x[:, :DW], x[:, DW:], er)
    ha = _tc_self(x, W_self, b.reshape(1, D))
    out = _tc_head(ha, parts, W_nbr, w_out.reshape(1, D))
    return out[:, 0]
